# Initial kernel scaffold; baseline (speedup 1.0000x reference)
import jax
import jax.numpy as jnp
from jax.experimental import pallas as pl


def kernel(t, pos, idcs_airfoil, velocity_in, W1, b1, Wm0, Wm, Wm2, Wg, We):
    raise NotImplementedError("write your pallas kernel here")


# speedup vs baseline: 7.9419x; 7.9419x over previous
import jax
import jax.numpy as jnp
from jax.experimental import pallas as pl

_VEL_MEAN = (37.750118255615234, 0.5372318625450134, 2.009599447250366)
_VEL_STD = (19.8649845123291, 7.343273639678955, 9.551141738891602)
_POS_MEAN = (0.8507418036460876, -6.422636200653642e-09, 0.37120404839515686)
_POS_STD = (0.40274253487586975, 0.07883177697658539, 0.2320450097322464)
_WALL_SCALE = 0.28871151953935625
_VORT_SCALE = 10.57309174537657

_NP = 10240
_RT = 128
_PADPOS = 1.0e4


def _feat_kernel(pos8_ref, wposT_ref, idx_ref, featx_ref, w1x_ref, w1m_ref,
                 h_ref, aux_ref):
    i = pl.program_id(1)
    pos8 = pos8_ref[0]
    xr = pos8[:, 3:4]
    yr = pos8[:, 4:5]
    zr = pos8[:, 5:6]
    wT = wposT_ref[0]
    d2w = ((xr - wT[0:1, :]) ** 2 + (yr - wT[1:2, :]) ** 2
           + (zr - wT[2:3, :]) ** 2)
    wall = jnp.sqrt(jnp.min(d2w, axis=1, keepdims=True) + 1e-8)
    wall_s = wall / _WALL_SCALE
    vort = jnp.exp(-wall * _VORT_SCALE)

    m_pts = wT.shape[1]
    row0 = i * _RT
    rowid = row0 + jax.lax.broadcasted_iota(jnp.int32, (_RT, m_pts), 0)
    idxv = idx_ref[0]
    af = jnp.max(jnp.where(rowid == idxv, 1.0, 0.0), axis=1, keepdims=True)

    fx = featx_ref[0]
    h = jnp.dot(fx, w1x_ref[...], preferred_element_type=jnp.float32)
    h = h + wall_s * w1m_ref[0:1, :] + af * w1m_ref[1:2, :] + w1m_ref[2:3, :]
    h_ref[0] = jnp.maximum(h, 0.0)

    aux = jnp.concatenate(
        [pos8[:, 0:3], wall_s, af, vort,
         jnp.zeros((_RT, 2), jnp.float32)], axis=1)
    aux_ref[0] = aux


def _knn_kernel(aux_ref, posT_ref, hfull_ref, htile_ref, wm0_ref, wm_ref,
                wm2_ref, wgp_ref, wef_ref, cst_ref, out_ref):
    i = pl.program_id(1)
    aux = aux_ref[0]
    xn = aux[:, 0:1]
    yn = aux[:, 1:2]
    zn = aux[:, 2:3]
    wall_s = aux[:, 3:4]
    af = aux[:, 4:5]
    vort = aux[:, 5:6]

    pT = posT_ref[0]
    d2 = ((xn - pT[0:1, :]) ** 2 + (yn - pT[1:2, :]) ** 2
          + (zn - pT[2:3, :]) ** 2)
    row0 = i * _RT
    colid = jax.lax.broadcasted_iota(jnp.int32, (_RT, _NP), 1)
    rowid = row0 + jax.lax.broadcasted_iota(jnp.int32, (_RT, _NP), 0)
    d2 = jnp.where(colid == rowid, 1e30, d2)

    inf = jnp.float32(3.0e38)

    def body(tt, carry):
        work, t16, t32 = carry
        m = jnp.min(work, axis=1, keepdims=True)
        t16 = jnp.where(tt == 15, m, t16)
        t32 = jnp.where(tt == 31, m, t32)
        work = jnp.where(work <= m, inf, work)
        return work, t16, t32

    zero = jnp.zeros((_RT, 1), jnp.float32)
    _, t16, t32 = jax.lax.fori_loop(0, 32, body, (d2, zero, zero))

    m16 = (d2 <= t16).astype(jnp.float32)
    m32 = (d2 <= t32).astype(jnp.float32)
    h = hfull_ref[0]
    s16 = jnp.dot(m16, h, preferred_element_type=jnp.float32)
    s32 = jnp.dot(m32, h, preferred_element_type=jnp.float32)
    c16 = jnp.sum(m16, axis=1, keepdims=True)
    c32 = jnp.sum(m32, axis=1, keepdims=True)
    a = s16 / jnp.maximum(c16, 1.0)
    ad = s32 / jnp.maximum(c32, 1.0)

    ht = htile_ref[0]
    h2 = jnp.dot(ht, wm0_ref[...], preferred_element_type=jnp.float32)
    h2 = h2 + jnp.dot(a, wm_ref[...], preferred_element_type=jnp.float32)
    h2 = h2 + jnp.dot(ad - a, wm2_ref[...], preferred_element_type=jnp.float32)
    h2 = jnp.maximum(h2, 0.0)

    wf = jnp.concatenate([wall_s, vort, jnp.zeros((_RT, 6), jnp.float32)],
                         axis=1)
    gp = jnp.dot(wf, wgp_ref[...], preferred_element_type=jnp.float32)
    g = gp[:, 0:4]
    g = g - jnp.max(g, axis=1, keepdims=True)
    g = jnp.exp(g)
    g = g / jnp.sum(g, axis=1, keepdims=True)

    p = jnp.dot(h2, wef_ref[...], preferred_element_type=jnp.float32)
    o = (g[:, 0:1] * p[:, 0:16] + g[:, 1:2] * p[:, 16:32]
         + g[:, 2:3] * p[:, 32:48] + g[:, 3:4] * p[:, 48:64])
    cst = cst_ref[...]
    o = o * cst[0:1, :] + cst[1:2, :]
    o = jnp.where(af > 0.0, 0.0, o)
    out_ref[0] = o


def kernel(t, pos, idcs_airfoil, velocity_in, W1, b1, Wm0, Wm, Wm2, Wg, We):
    del t
    B, N, _ = pos.shape
    T = velocity_in.shape[1]
    M = idcs_airfoil.shape[1]
    nt = _NP // _RT

    pos_n = (pos - jnp.array(_POS_MEAN, jnp.float32)) / (
        jnp.array(_POS_STD, jnp.float32) + 1e-8)
    vel_n = (velocity_in - jnp.array(_VEL_MEAN, jnp.float32)) / (
        jnp.array(_VEL_STD, jnp.float32) + 1e-8)

    featx = jnp.transpose(vel_n, (0, 2, 1, 3)).reshape(B, N, T * 3)
    featx = jnp.pad(featx, ((0, 0), (0, _NP - N), (0, 16 - T * 3)))

    posn_pad = jnp.pad(pos_n, ((0, 0), (0, _NP - N), (0, 0)),
                       constant_values=_PADPOS)
    posr_pad = jnp.pad(pos, ((0, 0), (0, _NP - N), (0, 0)),
                       constant_values=_PADPOS)
    pos8 = jnp.concatenate(
        [posn_pad, posr_pad, jnp.zeros((B, _NP, 2), jnp.float32)], axis=2)
    posT = jnp.pad(jnp.transpose(posn_pad, (0, 2, 1)), ((0, 0), (0, 5), (0, 0)))

    idx = idcs_airfoil.astype(jnp.int32)
    wall_pts = jnp.take_along_axis(pos, idx[:, :, None], axis=1)
    wposT = jnp.pad(jnp.transpose(wall_pts, (0, 2, 1)), ((0, 0), (0, 5), (0, 0)))
    idx3 = idx.reshape(B, 1, idx.shape[1])

    w1x = jnp.pad(W1[:15], ((0, 1), (0, 0)))
    w1m = jnp.zeros((8, 64), jnp.float32).at[0].set(W1[15]).at[1].set(
        W1[16]).at[2].set(b1)
    wgp = jnp.zeros((8, 8), jnp.float32).at[0:2, 0:4].set(Wg)
    wef = jnp.transpose(jnp.pad(We, ((0, 0), (0, 0), (0, 1))),
                        (1, 0, 2)).reshape(64, 64)
    std16 = jnp.pad(jnp.tile(jnp.array(_VEL_STD, jnp.float32), T), (0, 1))
    mean16 = jnp.pad(jnp.tile(jnp.array(_VEL_MEAN, jnp.float32), T), (0, 1))
    cst = jnp.zeros((8, 16), jnp.float32).at[0].set(std16).at[1].set(mean16)

    h, aux = pl.pallas_call(
        _feat_kernel,
        grid=(B, nt),
        in_specs=[
            pl.BlockSpec((1, _RT, 8), lambda b, i: (b, i, 0)),
            pl.BlockSpec((1, 8, M), lambda b, i: (b, 0, 0)),
            pl.BlockSpec((1, 1, M), lambda b, i: (b, 0, 0)),
            pl.BlockSpec((1, _RT, 16), lambda b, i: (b, i, 0)),
            pl.BlockSpec((16, 64), lambda b, i: (0, 0)),
            pl.BlockSpec((8, 64), lambda b, i: (0, 0)),
        ],
        out_specs=[
            pl.BlockSpec((1, _RT, 64), lambda b, i: (b, i, 0)),
            pl.BlockSpec((1, _RT, 8), lambda b, i: (b, i, 0)),
        ],
        out_shape=[
            jax.ShapeDtypeStruct((B, _NP, 64), jnp.float32),
            jax.ShapeDtypeStruct((B, _NP, 8), jnp.float32),
        ],
    )(pos8, wposT, idx3, featx, w1x, w1m)

    outp = pl.pallas_call(
        _knn_kernel,
        grid=(B, nt),
        in_specs=[
            pl.BlockSpec((1, _RT, 8), lambda b, i: (b, i, 0)),
            pl.BlockSpec((1, 8, _NP), lambda b, i: (b, 0, 0)),
            pl.BlockSpec((1, _NP, 64), lambda b, i: (b, 0, 0)),
            pl.BlockSpec((1, _RT, 64), lambda b, i: (b, i, 0)),
            pl.BlockSpec((64, 64), lambda b, i: (0, 0)),
            pl.BlockSpec((64, 64), lambda b, i: (0, 0)),
            pl.BlockSpec((64, 64), lambda b, i: (0, 0)),
            pl.BlockSpec((8, 8), lambda b, i: (0, 0)),
            pl.BlockSpec((64, 64), lambda b, i: (0, 0)),
            pl.BlockSpec((8, 16), lambda b, i: (0, 0)),
        ],
        out_specs=pl.BlockSpec((1, _RT, 16), lambda b, i: (b, i, 0)),
        out_shape=jax.ShapeDtypeStruct((B, _NP, 16), jnp.float32),
    )(aux, posT, h, h, Wm0, Wm, Wm2, wgp, wef, cst)

    out = outp[:, :N, :T * 3].reshape(B, N, T, 3)
    return jnp.transpose(out, (0, 2, 1, 3))


# speedup vs baseline: 28.9042x; 3.6395x over previous
import jax
import jax.numpy as jnp
from jax.experimental import pallas as pl

_VEL_MEAN = (37.750118255615234, 0.5372318625450134, 2.009599447250366)
_VEL_STD = (19.8649845123291, 7.343273639678955, 9.551141738891602)
_POS_MEAN = (0.8507418036460876, -6.422636200653642e-09, 0.37120404839515686)
_POS_STD = (0.40274253487586975, 0.07883177697658539, 0.2320450097322464)
_WALL_SCALE = 0.28871151953935625
_VORT_SCALE = 10.57309174537657

_NP = 10240
_RT = 128
_PADPOS = 1.0e4


def _feat_kernel(pos8_ref, wposT_ref, idx_ref, featx_ref, w1x_ref, w1m_ref,
                 h_ref, hb_ref, aux_ref):
    i = pl.program_id(1)
    pos8 = pos8_ref[0]
    xr = pos8[:, 3:4]
    yr = pos8[:, 4:5]
    zr = pos8[:, 5:6]
    wT = wposT_ref[0]
    d2w = ((xr - wT[0:1, :]) ** 2 + (yr - wT[1:2, :]) ** 2
           + (zr - wT[2:3, :]) ** 2)
    wall = jnp.sqrt(jnp.min(d2w, axis=1, keepdims=True) + 1e-8)
    wall_s = wall / _WALL_SCALE
    vort = jnp.exp(-wall * _VORT_SCALE)

    m_pts = wT.shape[1]
    row0 = i * _RT
    rowid = row0 + jax.lax.broadcasted_iota(jnp.int32, (_RT, m_pts), 0)
    idxv = idx_ref[0]
    af = jnp.max(jnp.where(rowid == idxv, 1.0, 0.0), axis=1, keepdims=True)

    fx = featx_ref[0]
    h = jnp.dot(fx, w1x_ref[...], preferred_element_type=jnp.float32)
    h = h + wall_s * w1m_ref[0:1, :] + af * w1m_ref[1:2, :] + w1m_ref[2:3, :]
    h = jnp.maximum(h, 0.0)
    h_ref[0] = h
    hb_ref[0] = h.astype(jnp.bfloat16)

    aux = jnp.concatenate(
        [pos8[:, 0:3], wall_s, af, vort,
         jnp.zeros((_RT, 2), jnp.float32)], axis=1)
    aux_ref[0] = aux


def _knn_kernel(aux_ref, posT3_ref, hb_ref, htile_ref, wm0_ref, wm_ref,
                wm2_ref, wgp_ref, wef_ref, cst_ref, out_ref):
    i = pl.program_id(1)
    aux = aux_ref[0]
    xn = aux[:, 0:1]
    yn = aux[:, 1:2]
    zn = aux[:, 2:3]
    wall_s = aux[:, 3:4]
    af = aux[:, 4:5]
    vort = aux[:, 5:6]

    ng = 128
    nm = _NP // ng
    pT3 = posT3_ref[0]
    x3 = xn[:, :, None]
    y3 = yn[:, :, None]
    z3 = zn[:, :, None]
    d3 = ((x3 - pT3[0][None]) ** 2 + (y3 - pT3[1][None]) ** 2
          + (z3 - pT3[2][None]) ** 2)
    row0 = i * _RT
    col3 = (jax.lax.broadcasted_iota(jnp.int32, (_RT, nm, ng), 1) * ng
            + jax.lax.broadcasted_iota(jnp.int32, (_RT, nm, ng), 2))
    rowid3 = row0 + jax.lax.broadcasted_iota(jnp.int32, (_RT, nm, ng), 0)
    d3 = jnp.where(col3 == rowid3, 1e30, d3)

    inf = jnp.float32(3.0e38)

    work = d3
    cands = []
    for _ in range(8):
        m = jnp.min(work, axis=1, keepdims=True)
        cands.append(m)
        work = jnp.where(work <= m, inf, work)
    cand = jnp.concatenate(cands, axis=1).reshape(_RT, 8 * ng)

    def body(tt, carry):
        wk, t16, t32 = carry
        m = jnp.min(wk, axis=1, keepdims=True)
        t16 = jnp.where(tt == 15, m, t16)
        t32 = jnp.where(tt == 31, m, t32)
        wk = jnp.where(wk <= m, inf, wk)
        return wk, t16, t32

    zero = jnp.zeros((_RT, 1), jnp.float32)
    _, t16, t32 = jax.lax.fori_loop(0, 32, body, (cand, zero, zero))

    d2 = d3.reshape(_RT, _NP)
    b16 = d2 <= t16
    b32 = d2 <= t32
    m16 = b16.astype(jnp.bfloat16)
    m32 = b32.astype(jnp.bfloat16)
    hb = hb_ref[0]
    s16 = jnp.dot(m16, hb, preferred_element_type=jnp.float32)
    s32 = jnp.dot(m32, hb, preferred_element_type=jnp.float32)
    c16 = jnp.sum(b16.astype(jnp.float32), axis=1, keepdims=True)
    c32 = jnp.sum(b32.astype(jnp.float32), axis=1, keepdims=True)
    a = s16 / jnp.maximum(c16, 1.0)
    ad = s32 / jnp.maximum(c32, 1.0)

    ht = htile_ref[0]
    h2 = jnp.dot(ht, wm0_ref[...], preferred_element_type=jnp.float32)
    h2 = h2 + jnp.dot(a, wm_ref[...], preferred_element_type=jnp.float32)
    h2 = h2 + jnp.dot(ad - a, wm2_ref[...], preferred_element_type=jnp.float32)
    h2 = jnp.maximum(h2, 0.0)

    wf = jnp.concatenate([wall_s, vort, jnp.zeros((_RT, 6), jnp.float32)],
                         axis=1)
    gp = jnp.dot(wf, wgp_ref[...], preferred_element_type=jnp.float32)
    g = gp[:, 0:4]
    g = g - jnp.max(g, axis=1, keepdims=True)
    g = jnp.exp(g)
    g = g / jnp.sum(g, axis=1, keepdims=True)

    p = jnp.dot(h2, wef_ref[...], preferred_element_type=jnp.float32)
    o = (g[:, 0:1] * p[:, 0:16] + g[:, 1:2] * p[:, 16:32]
         + g[:, 2:3] * p[:, 32:48] + g[:, 3:4] * p[:, 48:64])
    cst = cst_ref[...]
    o = o * cst[0:1, :] + cst[1:2, :]
    o = jnp.where(af > 0.0, 0.0, o)
    out_ref[0] = o


def kernel(t, pos, idcs_airfoil, velocity_in, W1, b1, Wm0, Wm, Wm2, Wg, We):
    del t
    B, N, _ = pos.shape
    T = velocity_in.shape[1]
    M = idcs_airfoil.shape[1]
    nt = _NP // _RT

    pos_n = (pos - jnp.array(_POS_MEAN, jnp.float32)) / (
        jnp.array(_POS_STD, jnp.float32) + 1e-8)
    vel_n = (velocity_in - jnp.array(_VEL_MEAN, jnp.float32)) / (
        jnp.array(_VEL_STD, jnp.float32) + 1e-8)

    featx = jnp.transpose(vel_n, (0, 2, 1, 3)).reshape(B, N, T * 3)
    featx = jnp.pad(featx, ((0, 0), (0, _NP - N), (0, 16 - T * 3)))

    posn_pad = jnp.pad(pos_n, ((0, 0), (0, _NP - N), (0, 0)),
                       constant_values=_PADPOS)
    posr_pad = jnp.pad(pos, ((0, 0), (0, _NP - N), (0, 0)),
                       constant_values=_PADPOS)
    pos8 = jnp.concatenate(
        [posn_pad, posr_pad, jnp.zeros((B, _NP, 2), jnp.float32)], axis=2)
    posT = jnp.pad(jnp.transpose(posn_pad, (0, 2, 1)), ((0, 0), (0, 5), (0, 0)))
    posT3 = posT.reshape(B, 8, _NP // 128, 128)

    idx = idcs_airfoil.astype(jnp.int32)
    wall_pts = jnp.take_along_axis(pos, idx[:, :, None], axis=1)
    wposT = jnp.pad(jnp.transpose(wall_pts, (0, 2, 1)), ((0, 0), (0, 5), (0, 0)))
    idx3 = idx.reshape(B, 1, idx.shape[1])

    w1x = jnp.pad(W1[:15], ((0, 1), (0, 0)))
    w1m = jnp.zeros((8, 64), jnp.float32).at[0].set(W1[15]).at[1].set(
        W1[16]).at[2].set(b1)
    wgp = jnp.zeros((8, 8), jnp.float32).at[0:2, 0:4].set(Wg)
    wef = jnp.transpose(jnp.pad(We, ((0, 0), (0, 0), (0, 1))),
                        (1, 0, 2)).reshape(64, 64)
    std16 = jnp.pad(jnp.tile(jnp.array(_VEL_STD, jnp.float32), T), (0, 1))
    mean16 = jnp.pad(jnp.tile(jnp.array(_VEL_MEAN, jnp.float32), T), (0, 1))
    cst = jnp.zeros((8, 16), jnp.float32).at[0].set(std16).at[1].set(mean16)

    h, hb, aux = pl.pallas_call(
        _feat_kernel,
        grid=(B, nt),
        in_specs=[
            pl.BlockSpec((1, _RT, 8), lambda b, i: (b, i, 0)),
            pl.BlockSpec((1, 8, M), lambda b, i: (b, 0, 0)),
            pl.BlockSpec((1, 1, M), lambda b, i: (b, 0, 0)),
            pl.BlockSpec((1, _RT, 16), lambda b, i: (b, i, 0)),
            pl.BlockSpec((16, 64), lambda b, i: (0, 0)),
            pl.BlockSpec((8, 64), lambda b, i: (0, 0)),
        ],
        out_specs=[
            pl.BlockSpec((1, _RT, 64), lambda b, i: (b, i, 0)),
            pl.BlockSpec((1, _RT, 64), lambda b, i: (b, i, 0)),
            pl.BlockSpec((1, _RT, 8), lambda b, i: (b, i, 0)),
        ],
        out_shape=[
            jax.ShapeDtypeStruct((B, _NP, 64), jnp.float32),
            jax.ShapeDtypeStruct((B, _NP, 64), jnp.bfloat16),
            jax.ShapeDtypeStruct((B, _NP, 8), jnp.float32),
        ],
    )(pos8, wposT, idx3, featx, w1x, w1m)

    outp = pl.pallas_call(
        _knn_kernel,
        grid=(B, nt),
        in_specs=[
            pl.BlockSpec((1, _RT, 8), lambda b, i: (b, i, 0)),
            pl.BlockSpec((1, 8, _NP // 128, 128), lambda b, i: (b, 0, 0, 0)),
            pl.BlockSpec((1, _NP, 64), lambda b, i: (b, 0, 0)),
            pl.BlockSpec((1, _RT, 64), lambda b, i: (b, i, 0)),
            pl.BlockSpec((64, 64), lambda b, i: (0, 0)),
            pl.BlockSpec((64, 64), lambda b, i: (0, 0)),
            pl.BlockSpec((64, 64), lambda b, i: (0, 0)),
            pl.BlockSpec((8, 8), lambda b, i: (0, 0)),
            pl.BlockSpec((64, 64), lambda b, i: (0, 0)),
            pl.BlockSpec((8, 16), lambda b, i: (0, 0)),
        ],
        out_specs=pl.BlockSpec((1, _RT, 16), lambda b, i: (b, i, 0)),
        out_shape=jax.ShapeDtypeStruct((B, _NP, 16), jnp.float32),
    )(aux, posT3, hb, h, Wm0, Wm, Wm2, wgp, wef, cst)

    out = outp[:, :N, :T * 3].reshape(B, N, T, 3)
    return jnp.transpose(out, (0, 2, 1, 3))


# speedup vs baseline: 41.4905x; 1.4354x over previous
import functools

import jax
import jax.numpy as jnp
import numpy as np
from jax.experimental import pallas as pl
from jax.experimental.shard_map import shard_map
from jax.sharding import Mesh, PartitionSpec as P

_VEL_MEAN = (37.750118255615234, 0.5372318625450134, 2.009599447250366)
_VEL_STD = (19.8649845123291, 7.343273639678955, 9.551141738891602)
_POS_MEAN = (0.8507418036460876, -6.422636200653642e-09, 0.37120404839515686)
_POS_STD = (0.40274253487586975, 0.07883177697658539, 0.2320450097322464)
_WALL_SCALE = 0.28871151953935625
_VORT_SCALE = 10.57309174537657

_NP = 10240
_RT = 128
_PADPOS = 1.0e4


def _feat_kernel(pos8_ref, wposT_ref, idx_ref, featx_ref, w1x_ref, w1m_ref,
                 h_ref, hb_ref, aux_ref):
    pos8 = pos8_ref[0]
    xr = pos8[:, 3:4]
    yr = pos8[:, 4:5]
    zr = pos8[:, 5:6]
    wT = wposT_ref[0]
    d2w = ((xr - wT[0:1, :]) ** 2 + (yr - wT[1:2, :]) ** 2
           + (zr - wT[2:3, :]) ** 2)
    wall = jnp.sqrt(jnp.min(d2w, axis=1, keepdims=True) + 1e-8)
    wall_s = wall / _WALL_SCALE
    vort = jnp.exp(-wall * _VORT_SCALE)

    rid = pos8[:, 6:7]
    idxv = idx_ref[0]
    af = jnp.max(jnp.where(rid == idxv, 1.0, 0.0), axis=1, keepdims=True)

    fx = featx_ref[0]
    h = jnp.dot(fx, w1x_ref[...], preferred_element_type=jnp.float32)
    h = h + wall_s * w1m_ref[0:1, :] + af * w1m_ref[1:2, :] + w1m_ref[2:3, :]
    h = jnp.maximum(h, 0.0)
    h_ref[0] = h
    hb_ref[0] = h.astype(jnp.bfloat16)

    aux = jnp.concatenate(
        [pos8[:, 0:3], wall_s, af, vort, rid,
         jnp.zeros((_RT, 1), jnp.float32)], axis=1)
    aux_ref[0] = aux


def _knn_kernel(aux_ref, posT3_ref, hb_ref, htile_ref, wm0_ref, wm_ref,
                wm2_ref, wgp_ref, wef_ref, cst_ref, out_ref):
    aux = aux_ref[0]
    xn = aux[:, 0:1]
    yn = aux[:, 1:2]
    zn = aux[:, 2:3]
    wall_s = aux[:, 3:4]
    af = aux[:, 4:5]
    vort = aux[:, 5:6]
    rid = aux[:, 6:7]

    ng = 128
    nm = _NP // ng
    pT3 = posT3_ref[0]
    x3 = xn[:, :, None]
    y3 = yn[:, :, None]
    z3 = zn[:, :, None]
    d3 = ((x3 - pT3[0][None]) ** 2 + (y3 - pT3[1][None]) ** 2
          + (z3 - pT3[2][None]) ** 2)
    col3 = (jax.lax.broadcasted_iota(jnp.int32, (_RT, nm, ng), 1) * ng
            + jax.lax.broadcasted_iota(jnp.int32, (_RT, nm, ng), 2))
    rid3 = rid.astype(jnp.int32)[:, :, None]
    d3 = jnp.where(col3 == rid3, 1e30, d3)

    inf = jnp.float32(3.0e38)

    work = d3
    cands = []
    for _ in range(8):
        m = jnp.min(work, axis=1, keepdims=True)
        cands.append(m)
        work = jnp.where(work <= m, inf, work)
    cand = jnp.concatenate(cands, axis=1).reshape(_RT, 8 * ng)

    def body(tt, carry):
        wk, t16, t32 = carry
        m = jnp.min(wk, axis=1, keepdims=True)
        t16 = jnp.where(tt == 15, m, t16)
        t32 = jnp.where(tt == 31, m, t32)
        wk = jnp.where(wk <= m, inf, wk)
        return wk, t16, t32

    zero = jnp.zeros((_RT, 1), jnp.float32)
    _, t16, t32 = jax.lax.fori_loop(0, 32, body, (cand, zero, zero))

    d2 = d3.reshape(_RT, _NP)
    b16 = d2 <= t16
    b32 = d2 <= t32
    m16 = b16.astype(jnp.bfloat16)
    m32 = b32.astype(jnp.bfloat16)
    hb = hb_ref[0]
    s16 = jnp.dot(m16, hb, preferred_element_type=jnp.float32)
    s32 = jnp.dot(m32, hb, preferred_element_type=jnp.float32)
    c16 = jnp.sum(b16.astype(jnp.float32), axis=1, keepdims=True)
    c32 = jnp.sum(b32.astype(jnp.float32), axis=1, keepdims=True)
    a = s16 / jnp.maximum(c16, 1.0)
    ad = s32 / jnp.maximum(c32, 1.0)

    ht = htile_ref[0]
    h2 = jnp.dot(ht, wm0_ref[...], preferred_element_type=jnp.float32)
    h2 = h2 + jnp.dot(a, wm_ref[...], preferred_element_type=jnp.float32)
    h2 = h2 + jnp.dot(ad - a, wm2_ref[...], preferred_element_type=jnp.float32)
    h2 = jnp.maximum(h2, 0.0)

    wf = jnp.concatenate([wall_s, vort, jnp.zeros((_RT, 6), jnp.float32)],
                         axis=1)
    gp = jnp.dot(wf, wgp_ref[...], preferred_element_type=jnp.float32)
    g = gp[:, 0:4]
    g = g - jnp.max(g, axis=1, keepdims=True)
    g = jnp.exp(g)
    g = g / jnp.sum(g, axis=1, keepdims=True)

    p = jnp.dot(h2, wef_ref[...], preferred_element_type=jnp.float32)
    o = (g[:, 0:1] * p[:, 0:16] + g[:, 1:2] * p[:, 16:32]
         + g[:, 2:3] * p[:, 32:48] + g[:, 3:4] * p[:, 48:64])
    cst = cst_ref[...]
    o = o * cst[0:1, :] + cst[1:2, :]
    o = jnp.where(af > 0.0, 0.0, o)
    out_ref[0] = o


def kernel(t, pos, idcs_airfoil, velocity_in, W1, b1, Wm0, Wm, Wm2, Wg, We):
    del t
    B, N, _ = pos.shape
    T = velocity_in.shape[1]

    pos_n = (pos - jnp.array(_POS_MEAN, jnp.float32)) / (
        jnp.array(_POS_STD, jnp.float32) + 1e-8)
    vel_n = (velocity_in - jnp.array(_VEL_MEAN, jnp.float32)) / (
        jnp.array(_VEL_STD, jnp.float32) + 1e-8)

    featx = jnp.transpose(vel_n, (0, 2, 1, 3)).reshape(B, N, T * 3)
    featx = jnp.pad(featx, ((0, 0), (0, _NP - N), (0, 16 - T * 3)))

    posn_pad = jnp.pad(pos_n, ((0, 0), (0, _NP - N), (0, 0)),
                       constant_values=_PADPOS)
    posr_pad = jnp.pad(pos, ((0, 0), (0, _NP - N), (0, 0)),
                       constant_values=_PADPOS)
    rid = jnp.broadcast_to(
        jnp.arange(_NP, dtype=jnp.float32)[None, :, None], (B, _NP, 1))
    pos8 = jnp.concatenate(
        [posn_pad, posr_pad, rid, jnp.zeros((B, _NP, 1), jnp.float32)], axis=2)
    posT = jnp.pad(jnp.transpose(posn_pad, (0, 2, 1)), ((0, 0), (0, 5), (0, 0)))
    posT3 = posT.reshape(B, 8, _NP // 128, 128)

    idx = idcs_airfoil.astype(jnp.int32)
    wall_pts = jnp.take_along_axis(pos, idx[:, :, None], axis=1)
    wposT = jnp.pad(jnp.transpose(wall_pts, (0, 2, 1)), ((0, 0), (0, 5), (0, 0)))
    idx3 = idx.reshape(B, 1, idx.shape[1]).astype(jnp.float32)

    w1x = jnp.pad(W1[:15], ((0, 1), (0, 0)))
    w1m = jnp.zeros((8, 64), jnp.float32).at[0].set(W1[15]).at[1].set(
        W1[16]).at[2].set(b1)
    wgp = jnp.zeros((8, 8), jnp.float32).at[0:2, 0:4].set(Wg)
    wef = jnp.transpose(jnp.pad(We, ((0, 0), (0, 0), (0, 1))),
                        (1, 0, 2)).reshape(64, 64)
    std16 = jnp.pad(jnp.tile(jnp.array(_VEL_STD, jnp.float32), T), (0, 1))
    mean16 = jnp.pad(jnp.tile(jnp.array(_VEL_MEAN, jnp.float32), T), (0, 1))
    cst = jnp.zeros((8, 16), jnp.float32).at[0].set(std16).at[1].set(mean16)

    args = (pos8, wposT, idx3, featx, posT3, w1x, w1m, Wm0, Wm, Wm2, wgp,
            wef, cst)
    devs = jax.devices()
    if len(devs) >= 2:
        mesh = Mesh(np.asarray(devs[:2]), ('x',))
        rep = P()
        shard_rows = P(None, 'x', None)
        fn = shard_map(
            functools.partial(_forward, sharded=True), mesh=mesh,
            in_specs=(shard_rows, rep, rep, shard_rows, rep, rep, rep, rep,
                      rep, rep, rep, rep, rep),
            out_specs=shard_rows, check_rep=False)
        outp = fn(*args)
    else:
        outp = _forward(*args, sharded=False)

    out = outp[:, :N, :T * 3].reshape(B, N, T, 3)
    return jnp.transpose(out, (0, 2, 1, 3))


def _forward(pos8, wposT, idx3, featx, posT3, w1x, w1m, Wm0, Wm, Wm2, wgp,
             wef, cst, sharded):
    B = pos8.shape[0]
    nt = pos8.shape[1] // _RT
    M = wposT.shape[2]

    h, hb, aux = pl.pallas_call(
        _feat_kernel,
        grid=(B, nt),
        in_specs=[
            pl.BlockSpec((1, _RT, 8), lambda b, i: (b, i, 0)),
            pl.BlockSpec((1, 8, M), lambda b, i: (b, 0, 0)),
            pl.BlockSpec((1, 1, M), lambda b, i: (b, 0, 0)),
            pl.BlockSpec((1, _RT, 16), lambda b, i: (b, i, 0)),
            pl.BlockSpec((16, 64), lambda b, i: (0, 0)),
            pl.BlockSpec((8, 64), lambda b, i: (0, 0)),
        ],
        out_specs=[
            pl.BlockSpec((1, _RT, 64), lambda b, i: (b, i, 0)),
            pl.BlockSpec((1, _RT, 64), lambda b, i: (b, i, 0)),
            pl.BlockSpec((1, _RT, 8), lambda b, i: (b, i, 0)),
        ],
        out_shape=[
            jax.ShapeDtypeStruct((B, pos8.shape[1], 64), jnp.float32),
            jax.ShapeDtypeStruct((B, pos8.shape[1], 64), jnp.bfloat16),
            jax.ShapeDtypeStruct((B, pos8.shape[1], 8), jnp.float32),
        ],
    )(pos8, wposT, idx3, featx, w1x, w1m)

    if sharded:
        hb = jax.lax.all_gather(hb, 'x', axis=1, tiled=True)

    outp = pl.pallas_call(
        _knn_kernel,
        grid=(B, nt),
        in_specs=[
            pl.BlockSpec((1, _RT, 8), lambda b, i: (b, i, 0)),
            pl.BlockSpec((1, 8, _NP // 128, 128), lambda b, i: (b, 0, 0, 0)),
            pl.BlockSpec((1, _NP, 64), lambda b, i: (b, 0, 0)),
            pl.BlockSpec((1, _RT, 64), lambda b, i: (b, i, 0)),
            pl.BlockSpec((64, 64), lambda b, i: (0, 0)),
            pl.BlockSpec((64, 64), lambda b, i: (0, 0)),
            pl.BlockSpec((64, 64), lambda b, i: (0, 0)),
            pl.BlockSpec((8, 8), lambda b, i: (0, 0)),
            pl.BlockSpec((64, 64), lambda b, i: (0, 0)),
            pl.BlockSpec((8, 16), lambda b, i: (0, 0)),
        ],
        out_specs=pl.BlockSpec((1, _RT, 16), lambda b, i: (b, i, 0)),
        out_shape=jax.ShapeDtypeStruct((B, pos8.shape[1], 16), jnp.float32),
    )(aux, posT3, hb, h, Wm0, Wm, Wm2, wgp, wef, cst)
    return outp


# speedup vs baseline: 53.1159x; 1.2802x over previous
import functools

import jax
import jax.numpy as jnp
import numpy as np
from jax.experimental import pallas as pl
from jax.experimental.shard_map import shard_map
from jax.sharding import Mesh, PartitionSpec as P

_VEL_MEAN = (37.750118255615234, 0.5372318625450134, 2.009599447250366)
_VEL_STD = (19.8649845123291, 7.343273639678955, 9.551141738891602)
_POS_MEAN = (0.8507418036460876, -6.422636200653642e-09, 0.37120404839515686)
_POS_STD = (0.40274253487586975, 0.07883177697658539, 0.2320450097322464)
_WALL_SCALE = 0.28871151953935625
_VORT_SCALE = 10.57309174537657

_NP = 10240
_RT = 128
_PADPOS = 1.0e4


def _feat_kernel(pos8_ref, wposT_ref, idx_ref, featx_ref, w1x_ref, w1m_ref,
                 h_ref, hb_ref, aux_ref):
    pos8 = pos8_ref[0]
    xr = pos8[:, 3:4]
    yr = pos8[:, 4:5]
    zr = pos8[:, 5:6]
    wT = wposT_ref[0]
    d2w = ((xr - wT[0:1, :]) ** 2 + (yr - wT[1:2, :]) ** 2
           + (zr - wT[2:3, :]) ** 2)
    wall = jnp.sqrt(jnp.min(d2w, axis=1, keepdims=True) + 1e-8)
    wall_s = wall / _WALL_SCALE
    vort = jnp.exp(-wall * _VORT_SCALE)

    rid = pos8[:, 6:7]
    idxv = idx_ref[0]
    af = jnp.max(jnp.where(rid == idxv, 1.0, 0.0), axis=1, keepdims=True)

    fx = featx_ref[0]
    h = jnp.dot(fx, w1x_ref[...], preferred_element_type=jnp.float32)
    h = h + wall_s * w1m_ref[0:1, :] + af * w1m_ref[1:2, :] + w1m_ref[2:3, :]
    h = jnp.maximum(h, 0.0)
    h_ref[0] = h
    hb_ref[0] = h.astype(jnp.bfloat16)

    aux = jnp.concatenate(
        [pos8[:, 0:3], wall_s, af, vort, rid,
         jnp.zeros((_RT, 1), jnp.float32)], axis=1)
    aux_ref[0] = aux


def _knn_kernel(aux_ref, posT3_ref, hb_ref, htile_ref, wm0_ref, wm_ref,
                wm2_ref, wgp_ref, wef_ref, cst_ref, out_ref):
    aux = aux_ref[0]
    xn = aux[:, 0:1]
    yn = aux[:, 1:2]
    zn = aux[:, 2:3]
    wall_s = aux[:, 3:4]
    af = aux[:, 4:5]
    vort = aux[:, 5:6]
    rid = aux[:, 6:7]

    ng = 128
    nm = _NP // ng
    pT3 = posT3_ref[0]
    x3 = xn[:, :, None]
    y3 = yn[:, :, None]
    z3 = zn[:, :, None]
    d3 = ((x3 - pT3[0][None]) ** 2 + (y3 - pT3[1][None]) ** 2
          + (z3 - pT3[2][None]) ** 2)
    col3 = (jax.lax.broadcasted_iota(jnp.int32, (_RT, nm, ng), 1) * ng
            + jax.lax.broadcasted_iota(jnp.int32, (_RT, nm, ng), 2))
    rid3 = rid.astype(jnp.int32)[:, :, None]
    d3 = jnp.where(col3 == rid3, 1e30, d3)

    inf = jnp.float32(3.0e38)

    work = d3
    cands = []
    for _ in range(8):
        m = jnp.min(work, axis=1, keepdims=True)
        cands.append(m)
        work = jnp.where(work <= m, inf, work)
    cand = jnp.concatenate(cands, axis=1).reshape(_RT, 8 * ng)

    def body(tt, carry):
        wk, t16, t32 = carry
        m = jnp.min(wk, axis=1, keepdims=True)
        t16 = jnp.where(tt == 15, m, t16)
        t32 = jnp.where(tt == 31, m, t32)
        wk = jnp.where(wk <= m, inf, wk)
        return wk, t16, t32

    zero = jnp.zeros((_RT, 1), jnp.float32)
    _, t16, t32 = jax.lax.fori_loop(0, 32, body, (cand, zero, zero))

    d2 = d3.reshape(_RT, _NP)
    b16 = d2 <= t16
    b32 = d2 <= t32
    m16 = b16.astype(jnp.bfloat16)
    m32 = b32.astype(jnp.bfloat16)
    hb = hb_ref[0]
    s16 = jnp.dot(m16, hb, preferred_element_type=jnp.float32)
    s32 = jnp.dot(m32, hb, preferred_element_type=jnp.float32)
    c16 = jnp.sum(b16.astype(jnp.float32), axis=1, keepdims=True)
    c32 = jnp.sum(b32.astype(jnp.float32), axis=1, keepdims=True)
    a = s16 / jnp.maximum(c16, 1.0)
    ad = s32 / jnp.maximum(c32, 1.0)

    ht = htile_ref[0]
    h2 = jnp.dot(ht, wm0_ref[...], preferred_element_type=jnp.float32)
    h2 = h2 + jnp.dot(a, wm_ref[...], preferred_element_type=jnp.float32)
    h2 = h2 + jnp.dot(ad - a, wm2_ref[...], preferred_element_type=jnp.float32)
    h2 = jnp.maximum(h2, 0.0)

    wf = jnp.concatenate([wall_s, vort, jnp.zeros((_RT, 6), jnp.float32)],
                         axis=1)
    gp = jnp.dot(wf, wgp_ref[...], preferred_element_type=jnp.float32)
    g = gp[:, 0:4]
    g = g - jnp.max(g, axis=1, keepdims=True)
    g = jnp.exp(g)
    g = g / jnp.sum(g, axis=1, keepdims=True)

    p = jnp.dot(h2, wef_ref[...], preferred_element_type=jnp.float32)
    o = (g[:, 0:1] * p[:, 0:16] + g[:, 1:2] * p[:, 16:32]
         + g[:, 2:3] * p[:, 32:48] + g[:, 3:4] * p[:, 48:64])
    cst = cst_ref[...]
    o = o * cst[0:1, :] + cst[1:2, :]
    o = jnp.where(af > 0.0, 0.0, o)
    out_ref[0] = o


def kernel(t, pos, idcs_airfoil, velocity_in, W1, b1, Wm0, Wm, Wm2, Wg, We):
    del t
    B, N, _ = pos.shape
    T = velocity_in.shape[1]

    pos_n = (pos - jnp.array(_POS_MEAN, jnp.float32)) / (
        jnp.array(_POS_STD, jnp.float32) + 1e-8)
    vel_n = (velocity_in - jnp.array(_VEL_MEAN, jnp.float32)) / (
        jnp.array(_VEL_STD, jnp.float32) + 1e-8)

    featx = jnp.transpose(vel_n, (0, 2, 1, 3)).reshape(B, N, T * 3)
    featx = jnp.pad(featx, ((0, 0), (0, _NP - N), (0, 16 - T * 3)))

    posn_pad = jnp.pad(pos_n, ((0, 0), (0, _NP - N), (0, 0)),
                       constant_values=_PADPOS)
    posr_pad = jnp.pad(pos, ((0, 0), (0, _NP - N), (0, 0)),
                       constant_values=_PADPOS)
    rid = jnp.broadcast_to(
        jnp.arange(_NP, dtype=jnp.float32)[None, :, None], (B, _NP, 1))
    pos8 = jnp.concatenate(
        [posn_pad, posr_pad, rid, jnp.zeros((B, _NP, 1), jnp.float32)], axis=2)
    posT = jnp.pad(jnp.transpose(posn_pad, (0, 2, 1)), ((0, 0), (0, 5), (0, 0)))
    posT3 = posT.reshape(B, 8, _NP // 128, 128)

    idx = idcs_airfoil.astype(jnp.int32)
    wall_pts = jnp.take_along_axis(pos, idx[:, :, None], axis=1)
    wposT = jnp.pad(jnp.transpose(wall_pts, (0, 2, 1)), ((0, 0), (0, 5), (0, 0)))
    idx3 = idx.reshape(B, 1, idx.shape[1]).astype(jnp.float32)

    w1x = jnp.pad(W1[:15], ((0, 1), (0, 0)))
    w1m = jnp.zeros((8, 64), jnp.float32).at[0].set(W1[15]).at[1].set(
        W1[16]).at[2].set(b1)
    wgp = jnp.zeros((8, 8), jnp.float32).at[0:2, 0:4].set(Wg)
    wef = jnp.transpose(jnp.pad(We, ((0, 0), (0, 0), (0, 1))),
                        (1, 0, 2)).reshape(64, 64)
    std16 = jnp.pad(jnp.tile(jnp.array(_VEL_STD, jnp.float32), T), (0, 1))
    mean16 = jnp.pad(jnp.tile(jnp.array(_VEL_MEAN, jnp.float32), T), (0, 1))
    cst = jnp.zeros((8, 16), jnp.float32).at[0].set(std16).at[1].set(mean16)

    args = (pos8, wposT, idx3, featx, posT3, w1x, w1m, Wm0, Wm, Wm2, wgp,
            wef, cst)
    devs = jax.devices()
    if len(devs) >= 2:
        mesh = Mesh(np.asarray(devs[:2]), ('x',))
        rep = P()
        fn = shard_map(
            functools.partial(_forward, sharded=True), mesh=mesh,
            in_specs=(rep,) * 13,
            out_specs=P(None, 'x', None), check_rep=False)
        outp = fn(*args)
    else:
        outp = _forward(*args, sharded=False)

    out = outp[:, :N, :T * 3].reshape(B, N, T, 3)
    return jnp.transpose(out, (0, 2, 1, 3))


def _forward(pos8, wposT, idx3, featx, posT3, w1x, w1m, Wm0, Wm, Wm2, wgp,
             wef, cst, sharded):
    B = pos8.shape[0]
    nt = pos8.shape[1] // _RT
    M = wposT.shape[2]

    h, hb, aux = pl.pallas_call(
        _feat_kernel,
        grid=(B, nt),
        in_specs=[
            pl.BlockSpec((1, _RT, 8), lambda b, i: (b, i, 0)),
            pl.BlockSpec((1, 8, M), lambda b, i: (b, 0, 0)),
            pl.BlockSpec((1, 1, M), lambda b, i: (b, 0, 0)),
            pl.BlockSpec((1, _RT, 16), lambda b, i: (b, i, 0)),
            pl.BlockSpec((16, 64), lambda b, i: (0, 0)),
            pl.BlockSpec((8, 64), lambda b, i: (0, 0)),
        ],
        out_specs=[
            pl.BlockSpec((1, _RT, 64), lambda b, i: (b, i, 0)),
            pl.BlockSpec((1, _RT, 64), lambda b, i: (b, i, 0)),
            pl.BlockSpec((1, _RT, 8), lambda b, i: (b, i, 0)),
        ],
        out_shape=[
            jax.ShapeDtypeStruct((B, pos8.shape[1], 64), jnp.float32),
            jax.ShapeDtypeStruct((B, pos8.shape[1], 64), jnp.bfloat16),
            jax.ShapeDtypeStruct((B, pos8.shape[1], 8), jnp.float32),
        ],
    )(pos8, wposT, idx3, featx, w1x, w1m)

    if sharded:
        npl = _NP // jax.lax.axis_size('x')
        off = jax.lax.axis_index('x') * npl
        aux = jax.lax.dynamic_slice_in_dim(aux, off, npl, axis=1)
        h = jax.lax.dynamic_slice_in_dim(h, off, npl, axis=1)
        nt = npl // _RT

    outp = pl.pallas_call(
        _knn_kernel,
        grid=(B, nt),
        in_specs=[
            pl.BlockSpec((1, _RT, 8), lambda b, i: (b, i, 0)),
            pl.BlockSpec((1, 8, _NP // 128, 128), lambda b, i: (b, 0, 0, 0)),
            pl.BlockSpec((1, _NP, 64), lambda b, i: (b, 0, 0)),
            pl.BlockSpec((1, _RT, 64), lambda b, i: (b, i, 0)),
            pl.BlockSpec((64, 64), lambda b, i: (0, 0)),
            pl.BlockSpec((64, 64), lambda b, i: (0, 0)),
            pl.BlockSpec((64, 64), lambda b, i: (0, 0)),
            pl.BlockSpec((8, 8), lambda b, i: (0, 0)),
            pl.BlockSpec((64, 64), lambda b, i: (0, 0)),
            pl.BlockSpec((8, 16), lambda b, i: (0, 0)),
        ],
        out_specs=pl.BlockSpec((1, _RT, 16), lambda b, i: (b, i, 0)),
        out_shape=jax.ShapeDtypeStruct((B, aux.shape[1], 16), jnp.float32),
    )(aux, posT3, hb, h, Wm0, Wm, Wm2, wgp, wef, cst)
    return outp


# speedup vs baseline: 53.5408x; 1.0080x over previous
import functools

import jax
import jax.numpy as jnp
import numpy as np
from jax.experimental import pallas as pl
from jax.experimental.shard_map import shard_map
from jax.sharding import Mesh, PartitionSpec as P

_VEL_MEAN = (37.750118255615234, 0.5372318625450134, 2.009599447250366)
_VEL_STD = (19.8649845123291, 7.343273639678955, 9.551141738891602)
_POS_MEAN = (0.8507418036460876, -6.422636200653642e-09, 0.37120404839515686)
_POS_STD = (0.40274253487586975, 0.07883177697658539, 0.2320450097322464)
_WALL_SCALE = 0.28871151953935625
_VORT_SCALE = 10.57309174537657

_NP = 10240
_RT = 128
_PADPOS = 1.0e4


def _feat_kernel(pos8_ref, wposT_ref, idx_ref, featx_ref, w1x_ref, w1m_ref,
                 h_ref, hb_ref, aux_ref):
    pos8 = pos8_ref[0]
    xr = pos8[:, 3:4]
    yr = pos8[:, 4:5]
    zr = pos8[:, 5:6]
    wT = wposT_ref[0]
    d2w = ((xr - wT[0:1, :]) ** 2 + (yr - wT[1:2, :]) ** 2
           + (zr - wT[2:3, :]) ** 2)
    wall = jnp.sqrt(jnp.min(d2w, axis=1, keepdims=True) + 1e-8)
    wall_s = wall / _WALL_SCALE
    vort = jnp.exp(-wall * _VORT_SCALE)

    rid = pos8[:, 6:7]
    idxv = idx_ref[0]
    af = jnp.max(jnp.where(rid == idxv, 1.0, 0.0), axis=1, keepdims=True)

    fx = featx_ref[0]
    h = jnp.dot(fx, w1x_ref[...], preferred_element_type=jnp.float32)
    h = h + wall_s * w1m_ref[0:1, :] + af * w1m_ref[1:2, :] + w1m_ref[2:3, :]
    h = jnp.maximum(h, 0.0)
    h_ref[0] = h
    hb_ref[0] = h.astype(jnp.bfloat16)

    aux = jnp.concatenate(
        [pos8[:, 0:3], wall_s, af, vort, rid,
         jnp.zeros((_RT, 1), jnp.float32)], axis=1)
    aux_ref[0] = aux


def _knn_kernel(aux_ref, posT3_ref, hb_ref, htile_ref, wm0_ref, wm_ref,
                wm2_ref, wgp_ref, wef_ref, cst_ref, out_ref):
    aux = aux_ref[0]
    xn = aux[:, 0:1]
    yn = aux[:, 1:2]
    zn = aux[:, 2:3]
    wall_s = aux[:, 3:4]
    af = aux[:, 4:5]
    vort = aux[:, 5:6]
    rid = aux[:, 6:7]

    ng = 128
    nm = _NP // ng
    pT3 = posT3_ref[0]
    x3 = xn[:, :, None]
    y3 = yn[:, :, None]
    z3 = zn[:, :, None]
    d3 = ((x3 - pT3[0][None]) ** 2 + (y3 - pT3[1][None]) ** 2
          + (z3 - pT3[2][None]) ** 2)
    col3 = (jax.lax.broadcasted_iota(jnp.int32, (_RT, nm, ng), 1) * ng
            + jax.lax.broadcasted_iota(jnp.int32, (_RT, nm, ng), 2))
    rid3 = rid.astype(jnp.int32)[:, :, None]
    d3 = jnp.where(col3 == rid3, 1e30, d3)

    inf = jnp.float32(3.0e38)

    cands = []
    v = jnp.full((_RT, 1, ng), -1.0, jnp.float32)
    for _ in range(8):
        m = jnp.min(jnp.where(d3 > v, d3, inf), axis=1, keepdims=True)
        cands.append(m)
        v = m
    cand = jnp.concatenate(cands, axis=1).reshape(_RT, 8 * ng)

    def body(tt, carry):
        v, t16, t32 = carry
        m = jnp.min(jnp.where(cand > v, cand, inf), axis=1, keepdims=True)
        t16 = jnp.where(tt == 15, m, t16)
        t32 = jnp.where(tt == 31, m, t32)
        return m, t16, t32

    zero = jnp.zeros((_RT, 1), jnp.float32)
    neg = jnp.full((_RT, 1), -1.0, jnp.float32)
    _, t16, t32 = jax.lax.fori_loop(0, 32, body, (neg, zero, zero))

    d2 = d3.reshape(_RT, _NP)
    b16 = d2 <= t16
    b32 = d2 <= t32
    m16 = b16.astype(jnp.bfloat16)
    m32 = b32.astype(jnp.bfloat16)
    hb = hb_ref[0]
    s16 = jnp.dot(m16, hb, preferred_element_type=jnp.float32)
    s32 = jnp.dot(m32, hb, preferred_element_type=jnp.float32)
    c16 = jnp.sum(b16.astype(jnp.float32), axis=1, keepdims=True)
    c32 = jnp.sum(b32.astype(jnp.float32), axis=1, keepdims=True)
    a = s16 / jnp.maximum(c16, 1.0)
    ad = s32 / jnp.maximum(c32, 1.0)

    ht = htile_ref[0]
    h2 = jnp.dot(ht, wm0_ref[...], preferred_element_type=jnp.float32)
    h2 = h2 + jnp.dot(a, wm_ref[...], preferred_element_type=jnp.float32)
    h2 = h2 + jnp.dot(ad - a, wm2_ref[...], preferred_element_type=jnp.float32)
    h2 = jnp.maximum(h2, 0.0)

    wf = jnp.concatenate([wall_s, vort, jnp.zeros((_RT, 6), jnp.float32)],
                         axis=1)
    gp = jnp.dot(wf, wgp_ref[...], preferred_element_type=jnp.float32)
    g = gp[:, 0:4]
    g = g - jnp.max(g, axis=1, keepdims=True)
    g = jnp.exp(g)
    g = g / jnp.sum(g, axis=1, keepdims=True)

    p = jnp.dot(h2, wef_ref[...], preferred_element_type=jnp.float32)
    o = (g[:, 0:1] * p[:, 0:16] + g[:, 1:2] * p[:, 16:32]
         + g[:, 2:3] * p[:, 32:48] + g[:, 3:4] * p[:, 48:64])
    cst = cst_ref[...]
    o = o * cst[0:1, :] + cst[1:2, :]
    o = jnp.where(af > 0.0, 0.0, o)
    out_ref[0] = o


def kernel(t, pos, idcs_airfoil, velocity_in, W1, b1, Wm0, Wm, Wm2, Wg, We):
    del t
    B, N, _ = pos.shape
    T = velocity_in.shape[1]

    pos_n = (pos - jnp.array(_POS_MEAN, jnp.float32)) / (
        jnp.array(_POS_STD, jnp.float32) + 1e-8)
    vel_n = (velocity_in - jnp.array(_VEL_MEAN, jnp.float32)) / (
        jnp.array(_VEL_STD, jnp.float32) + 1e-8)

    featx = jnp.transpose(vel_n, (0, 2, 1, 3)).reshape(B, N, T * 3)
    featx = jnp.pad(featx, ((0, 0), (0, _NP - N), (0, 16 - T * 3)))

    posn_pad = jnp.pad(pos_n, ((0, 0), (0, _NP - N), (0, 0)),
                       constant_values=_PADPOS)
    posr_pad = jnp.pad(pos, ((0, 0), (0, _NP - N), (0, 0)),
                       constant_values=_PADPOS)
    rid = jnp.broadcast_to(
        jnp.arange(_NP, dtype=jnp.float32)[None, :, None], (B, _NP, 1))
    pos8 = jnp.concatenate(
        [posn_pad, posr_pad, rid, jnp.zeros((B, _NP, 1), jnp.float32)], axis=2)
    posT = jnp.pad(jnp.transpose(posn_pad, (0, 2, 1)), ((0, 0), (0, 5), (0, 0)))
    posT3 = posT.reshape(B, 8, _NP // 128, 128)

    idx = idcs_airfoil.astype(jnp.int32)
    wall_pts = jnp.take_along_axis(pos, idx[:, :, None], axis=1)
    wposT = jnp.pad(jnp.transpose(wall_pts, (0, 2, 1)), ((0, 0), (0, 5), (0, 0)))
    idx3 = idx.reshape(B, 1, idx.shape[1]).astype(jnp.float32)

    w1x = jnp.pad(W1[:15], ((0, 1), (0, 0)))
    w1m = jnp.zeros((8, 64), jnp.float32).at[0].set(W1[15]).at[1].set(
        W1[16]).at[2].set(b1)
    wgp = jnp.zeros((8, 8), jnp.float32).at[0:2, 0:4].set(Wg)
    wef = jnp.transpose(jnp.pad(We, ((0, 0), (0, 0), (0, 1))),
                        (1, 0, 2)).reshape(64, 64)
    std16 = jnp.pad(jnp.tile(jnp.array(_VEL_STD, jnp.float32), T), (0, 1))
    mean16 = jnp.pad(jnp.tile(jnp.array(_VEL_MEAN, jnp.float32), T), (0, 1))
    cst = jnp.zeros((8, 16), jnp.float32).at[0].set(std16).at[1].set(mean16)

    args = (pos8, wposT, idx3, featx, posT3, w1x, w1m, Wm0, Wm, Wm2, wgp,
            wef, cst)
    devs = jax.devices()
    if len(devs) >= 2:
        mesh = Mesh(np.asarray(devs[:2]), ('x',))
        rep = P()
        fn = shard_map(
            functools.partial(_forward, sharded=True), mesh=mesh,
            in_specs=(rep,) * 13,
            out_specs=P(None, 'x', None), check_rep=False)
        outp = fn(*args)
    else:
        outp = _forward(*args, sharded=False)

    out = outp[:, :N, :T * 3].reshape(B, N, T, 3)
    return jnp.transpose(out, (0, 2, 1, 3))


def _forward(pos8, wposT, idx3, featx, posT3, w1x, w1m, Wm0, Wm, Wm2, wgp,
             wef, cst, sharded):
    B = pos8.shape[0]
    nt = pos8.shape[1] // _RT
    M = wposT.shape[2]

    h, hb, aux = pl.pallas_call(
        _feat_kernel,
        grid=(B, nt),
        in_specs=[
            pl.BlockSpec((1, _RT, 8), lambda b, i: (b, i, 0)),
            pl.BlockSpec((1, 8, M), lambda b, i: (b, 0, 0)),
            pl.BlockSpec((1, 1, M), lambda b, i: (b, 0, 0)),
            pl.BlockSpec((1, _RT, 16), lambda b, i: (b, i, 0)),
            pl.BlockSpec((16, 64), lambda b, i: (0, 0)),
            pl.BlockSpec((8, 64), lambda b, i: (0, 0)),
        ],
        out_specs=[
            pl.BlockSpec((1, _RT, 64), lambda b, i: (b, i, 0)),
            pl.BlockSpec((1, _RT, 64), lambda b, i: (b, i, 0)),
            pl.BlockSpec((1, _RT, 8), lambda b, i: (b, i, 0)),
        ],
        out_shape=[
            jax.ShapeDtypeStruct((B, pos8.shape[1], 64), jnp.float32),
            jax.ShapeDtypeStruct((B, pos8.shape[1], 64), jnp.bfloat16),
            jax.ShapeDtypeStruct((B, pos8.shape[1], 8), jnp.float32),
        ],
    )(pos8, wposT, idx3, featx, w1x, w1m)

    if sharded:
        npl = _NP // jax.lax.axis_size('x')
        off = jax.lax.axis_index('x') * npl
        aux = jax.lax.dynamic_slice_in_dim(aux, off, npl, axis=1)
        h = jax.lax.dynamic_slice_in_dim(h, off, npl, axis=1)
        nt = npl // _RT

    outp = pl.pallas_call(
        _knn_kernel,
        grid=(B, nt),
        in_specs=[
            pl.BlockSpec((1, _RT, 8), lambda b, i: (b, i, 0)),
            pl.BlockSpec((1, 8, _NP // 128, 128), lambda b, i: (b, 0, 0, 0)),
            pl.BlockSpec((1, _NP, 64), lambda b, i: (b, 0, 0)),
            pl.BlockSpec((1, _RT, 64), lambda b, i: (b, i, 0)),
            pl.BlockSpec((64, 64), lambda b, i: (0, 0)),
            pl.BlockSpec((64, 64), lambda b, i: (0, 0)),
            pl.BlockSpec((64, 64), lambda b, i: (0, 0)),
            pl.BlockSpec((8, 8), lambda b, i: (0, 0)),
            pl.BlockSpec((64, 64), lambda b, i: (0, 0)),
            pl.BlockSpec((8, 16), lambda b, i: (0, 0)),
        ],
        out_specs=pl.BlockSpec((1, _RT, 16), lambda b, i: (b, i, 0)),
        out_shape=jax.ShapeDtypeStruct((B, aux.shape[1], 16), jnp.float32),
    )(aux, posT3, hb, h, Wm0, Wm, Wm2, wgp, wef, cst)
    return outp


# speedup vs baseline: 53.8272x; 1.0053x over previous
import functools

import jax
import jax.numpy as jnp
import numpy as np
from jax.experimental import pallas as pl
from jax.experimental.shard_map import shard_map
from jax.sharding import Mesh, PartitionSpec as P

_VEL_MEAN = (37.750118255615234, 0.5372318625450134, 2.009599447250366)
_VEL_STD = (19.8649845123291, 7.343273639678955, 9.551141738891602)
_POS_MEAN = (0.8507418036460876, -6.422636200653642e-09, 0.37120404839515686)
_POS_STD = (0.40274253487586975, 0.07883177697658539, 0.2320450097322464)
_WALL_SCALE = 0.28871151953935625
_VORT_SCALE = 10.57309174537657

_NP = 10240
_RT = 128
_PADPOS = 1.0e4


def _feat_kernel(pos8_ref, wposT_ref, idx_ref, featx_ref, w1x_ref, w1m_ref,
                 h_ref, hb_ref, aux_ref):
    pos8 = pos8_ref[0]
    xr = pos8[:, 3:4]
    yr = pos8[:, 4:5]
    zr = pos8[:, 5:6]
    wT = wposT_ref[0]
    d2w = ((xr - wT[0:1, :]) ** 2 + (yr - wT[1:2, :]) ** 2
           + (zr - wT[2:3, :]) ** 2)
    wall = jnp.sqrt(jnp.min(d2w, axis=1, keepdims=True) + 1e-8)
    wall_s = wall / _WALL_SCALE
    vort = jnp.exp(-wall * _VORT_SCALE)

    rid = pos8[:, 6:7]
    idxv = idx_ref[0]
    af = jnp.max(jnp.where(rid == idxv, 1.0, 0.0), axis=1, keepdims=True)

    fx = featx_ref[0]
    h = jnp.dot(fx, w1x_ref[...], preferred_element_type=jnp.float32)
    h = h + wall_s * w1m_ref[0:1, :] + af * w1m_ref[1:2, :] + w1m_ref[2:3, :]
    h = jnp.maximum(h, 0.0)
    h_ref[0] = h
    hb_ref[0] = h.astype(jnp.bfloat16)

    aux = jnp.concatenate(
        [pos8[:, 0:3], wall_s, af, vort, rid,
         jnp.zeros((_RT, 1), jnp.float32)], axis=1)
    aux_ref[0] = aux


def _knn_kernel(aux_ref, posT3_ref, hb_ref, htile_ref, wm0_ref, wm_ref,
                wm2_ref, wgp_ref, wef_ref, cst_ref, out_ref):
    aux = aux_ref[0]
    xn = aux[:, 0:1]
    yn = aux[:, 1:2]
    zn = aux[:, 2:3]
    wall_s = aux[:, 3:4]
    af = aux[:, 4:5]
    vort = aux[:, 5:6]
    rid = aux[:, 6:7]

    ng = 128
    nm = _NP // ng
    pT3 = posT3_ref[0]
    x3 = xn[:, :, None]
    y3 = yn[:, :, None]
    z3 = zn[:, :, None]
    rs3 = (xn * xn + yn * yn + zn * zn)[:, :, None]
    d3 = ((x3 * pT3[0][None] + y3 * pT3[1][None])
          + (z3 * pT3[2][None] + pT3[3][None])) + rs3
    col3 = (jax.lax.broadcasted_iota(jnp.int32, (_RT, nm, ng), 1) * ng
            + jax.lax.broadcasted_iota(jnp.int32, (_RT, nm, ng), 2))
    rid3 = rid.astype(jnp.int32)[:, :, None]
    d3 = jnp.where(col3 == rid3, 1e30, d3)

    inf = jnp.float32(3.0e38)

    cands = []
    v = jnp.full((_RT, 1, ng), -1.0, jnp.float32)
    for _ in range(8):
        m = jnp.min(jnp.where(d3 > v, d3, inf), axis=1, keepdims=True)
        cands.append(m)
        v = m
    cand = jnp.concatenate(cands, axis=1).reshape(_RT, 8 * ng)

    def body(tt, carry):
        v, t16, t32 = carry
        m = jnp.min(jnp.where(cand > v, cand, inf), axis=1, keepdims=True)
        t16 = jnp.where(tt == 15, m, t16)
        t32 = jnp.where(tt == 31, m, t32)
        return m, t16, t32

    zero = jnp.zeros((_RT, 1), jnp.float32)
    neg = jnp.full((_RT, 1), -1.0, jnp.float32)
    _, t16, t32 = jax.lax.fori_loop(0, 32, body, (neg, zero, zero))

    d2 = d3.reshape(_RT, _NP)
    b16 = d2 <= t16
    b32 = d2 <= t32
    m16 = b16.astype(jnp.bfloat16)
    m32 = b32.astype(jnp.bfloat16)
    hb = hb_ref[0]
    s16 = jnp.dot(m16, hb, preferred_element_type=jnp.float32)
    s32 = jnp.dot(m32, hb, preferred_element_type=jnp.float32)
    c16 = jnp.sum(m16, axis=1, keepdims=True).astype(jnp.float32)
    c32 = jnp.sum(m32, axis=1, keepdims=True).astype(jnp.float32)
    a = s16 / jnp.maximum(c16, 1.0)
    ad = s32 / jnp.maximum(c32, 1.0)

    ht = htile_ref[0]
    h2 = jnp.dot(ht, wm0_ref[...], preferred_element_type=jnp.float32)
    h2 = h2 + jnp.dot(a, wm_ref[...], preferred_element_type=jnp.float32)
    h2 = h2 + jnp.dot(ad - a, wm2_ref[...], preferred_element_type=jnp.float32)
    h2 = jnp.maximum(h2, 0.0)

    wf = jnp.concatenate([wall_s, vort, jnp.zeros((_RT, 6), jnp.float32)],
                         axis=1)
    gp = jnp.dot(wf, wgp_ref[...], preferred_element_type=jnp.float32)
    g = gp[:, 0:4]
    g = g - jnp.max(g, axis=1, keepdims=True)
    g = jnp.exp(g)
    g = g / jnp.sum(g, axis=1, keepdims=True)

    p = jnp.dot(h2, wef_ref[...], preferred_element_type=jnp.float32)
    o = (g[:, 0:1] * p[:, 0:16] + g[:, 1:2] * p[:, 16:32]
         + g[:, 2:3] * p[:, 32:48] + g[:, 3:4] * p[:, 48:64])
    cst = cst_ref[...]
    o = o * cst[0:1, :] + cst[1:2, :]
    o = jnp.where(af > 0.0, 0.0, o)
    out_ref[0] = o


def kernel(t, pos, idcs_airfoil, velocity_in, W1, b1, Wm0, Wm, Wm2, Wg, We):
    del t
    B, N, _ = pos.shape
    T = velocity_in.shape[1]

    pos_n = (pos - jnp.array(_POS_MEAN, jnp.float32)) / (
        jnp.array(_POS_STD, jnp.float32) + 1e-8)
    vel_n = (velocity_in - jnp.array(_VEL_MEAN, jnp.float32)) / (
        jnp.array(_VEL_STD, jnp.float32) + 1e-8)

    featx = jnp.transpose(vel_n, (0, 2, 1, 3)).reshape(B, N, T * 3)
    featx = jnp.pad(featx, ((0, 0), (0, _NP - N), (0, 16 - T * 3)))

    posn_pad = jnp.pad(pos_n, ((0, 0), (0, _NP - N), (0, 0)),
                       constant_values=_PADPOS)
    posr_pad = jnp.pad(pos, ((0, 0), (0, _NP - N), (0, 0)),
                       constant_values=_PADPOS)
    rid = jnp.broadcast_to(
        jnp.arange(_NP, dtype=jnp.float32)[None, :, None], (B, _NP, 1))
    pos8 = jnp.concatenate(
        [posn_pad, posr_pad, rid, jnp.zeros((B, _NP, 1), jnp.float32)], axis=2)
    posT = jnp.transpose(posn_pad, (0, 2, 1))
    colsq = jnp.sum(posn_pad * posn_pad, axis=2)[:, None, :]
    posT8 = jnp.concatenate(
        [-2.0 * posT, colsq, jnp.zeros((B, 4, _NP), jnp.float32)], axis=1)
    posT3 = posT8.reshape(B, 8, _NP // 128, 128)

    idx = idcs_airfoil.astype(jnp.int32)
    wall_pts = jnp.take_along_axis(pos, idx[:, :, None], axis=1)
    wposT = jnp.pad(jnp.transpose(wall_pts, (0, 2, 1)), ((0, 0), (0, 5), (0, 0)))
    idx3 = idx.reshape(B, 1, idx.shape[1]).astype(jnp.float32)

    w1x = jnp.pad(W1[:15], ((0, 1), (0, 0)))
    w1m = jnp.zeros((8, 64), jnp.float32).at[0].set(W1[15]).at[1].set(
        W1[16]).at[2].set(b1)
    wgp = jnp.zeros((8, 8), jnp.float32).at[0:2, 0:4].set(Wg)
    wef = jnp.transpose(jnp.pad(We, ((0, 0), (0, 0), (0, 1))),
                        (1, 0, 2)).reshape(64, 64)
    std16 = jnp.pad(jnp.tile(jnp.array(_VEL_STD, jnp.float32), T), (0, 1))
    mean16 = jnp.pad(jnp.tile(jnp.array(_VEL_MEAN, jnp.float32), T), (0, 1))
    cst = jnp.zeros((8, 16), jnp.float32).at[0].set(std16).at[1].set(mean16)

    args = (pos8, wposT, idx3, featx, posT3, w1x, w1m, Wm0, Wm, Wm2, wgp,
            wef, cst)
    devs = jax.devices()
    if len(devs) >= 2:
        mesh = Mesh(np.asarray(devs[:2]), ('x',))
        rep = P()
        fn = shard_map(
            functools.partial(_forward, sharded=True), mesh=mesh,
            in_specs=(rep,) * 13,
            out_specs=P(None, 'x', None), check_rep=False)
        outp = fn(*args)
    else:
        outp = _forward(*args, sharded=False)

    out = outp[:, :N, :T * 3].reshape(B, N, T, 3)
    return jnp.transpose(out, (0, 2, 1, 3))


def _forward(pos8, wposT, idx3, featx, posT3, w1x, w1m, Wm0, Wm, Wm2, wgp,
             wef, cst, sharded):
    B = pos8.shape[0]
    nt = pos8.shape[1] // _RT
    M = wposT.shape[2]

    h, hb, aux = pl.pallas_call(
        _feat_kernel,
        grid=(B, nt),
        in_specs=[
            pl.BlockSpec((1, _RT, 8), lambda b, i: (b, i, 0)),
            pl.BlockSpec((1, 8, M), lambda b, i: (b, 0, 0)),
            pl.BlockSpec((1, 1, M), lambda b, i: (b, 0, 0)),
            pl.BlockSpec((1, _RT, 16), lambda b, i: (b, i, 0)),
            pl.BlockSpec((16, 64), lambda b, i: (0, 0)),
            pl.BlockSpec((8, 64), lambda b, i: (0, 0)),
        ],
        out_specs=[
            pl.BlockSpec((1, _RT, 64), lambda b, i: (b, i, 0)),
            pl.BlockSpec((1, _RT, 64), lambda b, i: (b, i, 0)),
            pl.BlockSpec((1, _RT, 8), lambda b, i: (b, i, 0)),
        ],
        out_shape=[
            jax.ShapeDtypeStruct((B, pos8.shape[1], 64), jnp.float32),
            jax.ShapeDtypeStruct((B, pos8.shape[1], 64), jnp.bfloat16),
            jax.ShapeDtypeStruct((B, pos8.shape[1], 8), jnp.float32),
        ],
    )(pos8, wposT, idx3, featx, w1x, w1m)

    if sharded:
        npl = _NP // jax.lax.axis_size('x')
        off = jax.lax.axis_index('x') * npl
        aux = jax.lax.dynamic_slice_in_dim(aux, off, npl, axis=1)
        h = jax.lax.dynamic_slice_in_dim(h, off, npl, axis=1)
        nt = npl // _RT

    outp = pl.pallas_call(
        _knn_kernel,
        grid=(B, nt),
        in_specs=[
            pl.BlockSpec((1, _RT, 8), lambda b, i: (b, i, 0)),
            pl.BlockSpec((1, 8, _NP // 128, 128), lambda b, i: (b, 0, 0, 0)),
            pl.BlockSpec((1, _NP, 64), lambda b, i: (b, 0, 0)),
            pl.BlockSpec((1, _RT, 64), lambda b, i: (b, i, 0)),
            pl.BlockSpec((64, 64), lambda b, i: (0, 0)),
            pl.BlockSpec((64, 64), lambda b, i: (0, 0)),
            pl.BlockSpec((64, 64), lambda b, i: (0, 0)),
            pl.BlockSpec((8, 8), lambda b, i: (0, 0)),
            pl.BlockSpec((64, 64), lambda b, i: (0, 0)),
            pl.BlockSpec((8, 16), lambda b, i: (0, 0)),
        ],
        out_specs=pl.BlockSpec((1, _RT, 16), lambda b, i: (b, i, 0)),
        out_shape=jax.ShapeDtypeStruct((B, aux.shape[1], 16), jnp.float32),
    )(aux, posT3, hb, h, Wm0, Wm, Wm2, wgp, wef, cst)
    return outp


# speedup vs baseline: 54.6275x; 1.0149x over previous
import functools

import jax
import jax.numpy as jnp
import numpy as np
from jax.experimental import pallas as pl
from jax.experimental.shard_map import shard_map
from jax.sharding import Mesh, PartitionSpec as P

_VEL_MEAN = (37.750118255615234, 0.5372318625450134, 2.009599447250366)
_VEL_STD = (19.8649845123291, 7.343273639678955, 9.551141738891602)
_POS_MEAN = (0.8507418036460876, -6.422636200653642e-09, 0.37120404839515686)
_POS_STD = (0.40274253487586975, 0.07883177697658539, 0.2320450097322464)
_WALL_SCALE = 0.28871151953935625
_VORT_SCALE = 10.57309174537657

_NP = 10240
_RT = 128
_PADPOS = 1.0e4


def _feat_kernel(pos8_ref, wposT_ref, idx_ref, featx_ref, w1x_ref, w1m_ref,
                 h_ref, hb_ref, aux_ref):
    pos8 = pos8_ref[0]
    xr = pos8[:, 3:4]
    yr = pos8[:, 4:5]
    zr = pos8[:, 5:6]
    wT = wposT_ref[0]
    d2w = ((xr - wT[0:1, :]) ** 2 + (yr - wT[1:2, :]) ** 2
           + (zr - wT[2:3, :]) ** 2)
    wall = jnp.sqrt(jnp.min(d2w, axis=1, keepdims=True) + 1e-8)
    wall_s = wall / _WALL_SCALE
    vort = jnp.exp(-wall * _VORT_SCALE)

    rid = pos8[:, 6:7]
    idxv = idx_ref[0]
    af = jnp.max(jnp.where(rid == idxv, 1.0, 0.0), axis=1, keepdims=True)

    fx = featx_ref[0]
    h = jnp.dot(fx, w1x_ref[...], preferred_element_type=jnp.float32)
    h = h + wall_s * w1m_ref[0:1, :] + af * w1m_ref[1:2, :] + w1m_ref[2:3, :]
    h = jnp.maximum(h, 0.0)
    h_ref[0] = h
    hb_ref[0] = h.astype(jnp.bfloat16)

    aux = jnp.concatenate(
        [pos8[:, 0:3], wall_s, af, vort, rid,
         jnp.zeros((_RT, 1), jnp.float32)], axis=1)
    aux_ref[0] = aux


def _knn_kernel(aux_ref, posT3_ref, hb_ref, htile_ref, wm0_ref, wm_ref,
                wm2_ref, wgp_ref, wef_ref, cst_ref, out_ref):
    aux = aux_ref[0]
    xn = aux[:, 0:1]
    yn = aux[:, 1:2]
    zn = aux[:, 2:3]
    wall_s = aux[:, 3:4]
    af = aux[:, 4:5]
    vort = aux[:, 5:6]
    rid = aux[:, 6:7]

    ng = 128
    nm = _NP // ng
    pT3 = posT3_ref[0]
    x3 = xn[:, :, None]
    y3 = yn[:, :, None]
    z3 = zn[:, :, None]
    d3 = ((x3 - pT3[0][None]) ** 2 + (y3 - pT3[1][None]) ** 2
          + (z3 - pT3[2][None]) ** 2)
    col3 = (jax.lax.broadcasted_iota(jnp.int32, (_RT, nm, ng), 1) * ng
            + jax.lax.broadcasted_iota(jnp.int32, (_RT, nm, ng), 2))
    rid3 = rid.astype(jnp.int32)[:, :, None]
    d3 = jnp.where(col3 == rid3, 1e30, d3)

    inf = jnp.float32(3.0e38)

    cands = []
    v = jnp.full((_RT, 1, ng), -1.0, jnp.float32)
    for _ in range(8):
        m = jnp.min(jnp.where(d3 > v, d3, inf), axis=1, keepdims=True)
        cands.append(m)
        v = m
    cand = jnp.concatenate(cands, axis=1).reshape(_RT, 8 * ng)

    def body(tt, carry):
        v, t16, t32 = carry
        m = jnp.min(jnp.where(cand > v, cand, inf), axis=1, keepdims=True)
        t16 = jnp.where(tt == 15, m, t16)
        t32 = jnp.where(tt == 31, m, t32)
        return m, t16, t32

    zero = jnp.zeros((_RT, 1), jnp.float32)
    neg = jnp.full((_RT, 1), -1.0, jnp.float32)
    _, t16, t32 = jax.lax.fori_loop(0, 32, body, (neg, zero, zero))

    d2 = d3.reshape(_RT, _NP)
    b16 = d2 <= t16
    b32 = d2 <= t32
    m16 = b16.astype(jnp.bfloat16)
    m32 = b32.astype(jnp.bfloat16)
    hb = hb_ref[0]
    s16 = jnp.dot(m16, hb, preferred_element_type=jnp.float32)
    s32 = jnp.dot(m32, hb, preferred_element_type=jnp.float32)
    c16 = jnp.sum(m16, axis=1, keepdims=True).astype(jnp.float32)
    c32 = jnp.sum(m32, axis=1, keepdims=True).astype(jnp.float32)
    a = s16 / jnp.maximum(c16, 1.0)
    ad = s32 / jnp.maximum(c32, 1.0)

    ht = htile_ref[0]
    h2 = jnp.dot(ht, wm0_ref[...], preferred_element_type=jnp.float32)
    h2 = h2 + jnp.dot(a, wm_ref[...], preferred_element_type=jnp.float32)
    h2 = h2 + jnp.dot(ad - a, wm2_ref[...], preferred_element_type=jnp.float32)
    h2 = jnp.maximum(h2, 0.0)

    wf = jnp.concatenate([wall_s, vort, jnp.zeros((_RT, 6), jnp.float32)],
                         axis=1)
    gp = jnp.dot(wf, wgp_ref[...], preferred_element_type=jnp.float32)
    g = gp[:, 0:4]
    g = g - jnp.max(g, axis=1, keepdims=True)
    g = jnp.exp(g)
    g = g / jnp.sum(g, axis=1, keepdims=True)

    p = jnp.dot(h2, wef_ref[...], preferred_element_type=jnp.float32)
    o = (g[:, 0:1] * p[:, 0:16] + g[:, 1:2] * p[:, 16:32]
         + g[:, 2:3] * p[:, 32:48] + g[:, 3:4] * p[:, 48:64])
    cst = cst_ref[...]
    o = o * cst[0:1, :] + cst[1:2, :]
    o = jnp.where(af > 0.0, 0.0, o)
    out_ref[0] = o


def kernel(t, pos, idcs_airfoil, velocity_in, W1, b1, Wm0, Wm, Wm2, Wg, We):
    del t
    B, N, _ = pos.shape
    T = velocity_in.shape[1]

    pos_n = (pos - jnp.array(_POS_MEAN, jnp.float32)) / (
        jnp.array(_POS_STD, jnp.float32) + 1e-8)
    vel_n = (velocity_in - jnp.array(_VEL_MEAN, jnp.float32)) / (
        jnp.array(_VEL_STD, jnp.float32) + 1e-8)

    featx = jnp.transpose(vel_n, (0, 2, 1, 3)).reshape(B, N, T * 3)
    featx = jnp.pad(featx, ((0, 0), (0, _NP - N), (0, 16 - T * 3)))

    posn_pad = jnp.pad(pos_n, ((0, 0), (0, _NP - N), (0, 0)),
                       constant_values=_PADPOS)
    posr_pad = jnp.pad(pos, ((0, 0), (0, _NP - N), (0, 0)),
                       constant_values=_PADPOS)
    rid = jnp.broadcast_to(
        jnp.arange(_NP, dtype=jnp.float32)[None, :, None], (B, _NP, 1))
    pos8 = jnp.concatenate(
        [posn_pad, posr_pad, rid, jnp.zeros((B, _NP, 1), jnp.float32)], axis=2)
    posT = jnp.pad(jnp.transpose(posn_pad, (0, 2, 1)), ((0, 0), (0, 5), (0, 0)))
    posT3 = posT.reshape(B, 8, _NP // 128, 128)

    idx = idcs_airfoil.astype(jnp.int32)
    wall_pts = jnp.take_along_axis(pos, idx[:, :, None], axis=1)
    wposT = jnp.pad(jnp.transpose(wall_pts, (0, 2, 1)), ((0, 0), (0, 5), (0, 0)))
    idx3 = idx.reshape(B, 1, idx.shape[1]).astype(jnp.float32)

    w1x = jnp.pad(W1[:15], ((0, 1), (0, 0)))
    w1m = jnp.zeros((8, 64), jnp.float32).at[0].set(W1[15]).at[1].set(
        W1[16]).at[2].set(b1)
    wgp = jnp.zeros((8, 8), jnp.float32).at[0:2, 0:4].set(Wg)
    wef = jnp.transpose(jnp.pad(We, ((0, 0), (0, 0), (0, 1))),
                        (1, 0, 2)).reshape(64, 64)
    std16 = jnp.pad(jnp.tile(jnp.array(_VEL_STD, jnp.float32), T), (0, 1))
    mean16 = jnp.pad(jnp.tile(jnp.array(_VEL_MEAN, jnp.float32), T), (0, 1))
    cst = jnp.zeros((8, 16), jnp.float32).at[0].set(std16).at[1].set(mean16)

    args = (pos8, wposT, idx3, featx, posT3, w1x, w1m, Wm0, Wm, Wm2, wgp,
            wef, cst)
    devs = jax.devices()
    if len(devs) >= 2:
        mesh = Mesh(np.asarray(devs[:2]), ('x',))
        rep = P()
        fn = shard_map(
            functools.partial(_forward, sharded=True), mesh=mesh,
            in_specs=(rep,) * 13,
            out_specs=P(None, 'x', None), check_rep=False)
        outp = fn(*args)
    else:
        outp = _forward(*args, sharded=False)

    out = outp[:, :N, :T * 3].reshape(B, N, T, 3)
    return jnp.transpose(out, (0, 2, 1, 3))


def _forward(pos8, wposT, idx3, featx, posT3, w1x, w1m, Wm0, Wm, Wm2, wgp,
             wef, cst, sharded):
    B = pos8.shape[0]
    nt = pos8.shape[1] // _RT
    M = wposT.shape[2]

    h, hb, aux = pl.pallas_call(
        _feat_kernel,
        grid=(B, nt),
        in_specs=[
            pl.BlockSpec((1, _RT, 8), lambda b, i: (b, i, 0)),
            pl.BlockSpec((1, 8, M), lambda b, i: (b, 0, 0)),
            pl.BlockSpec((1, 1, M), lambda b, i: (b, 0, 0)),
            pl.BlockSpec((1, _RT, 16), lambda b, i: (b, i, 0)),
            pl.BlockSpec((16, 64), lambda b, i: (0, 0)),
            pl.BlockSpec((8, 64), lambda b, i: (0, 0)),
        ],
        out_specs=[
            pl.BlockSpec((1, _RT, 64), lambda b, i: (b, i, 0)),
            pl.BlockSpec((1, _RT, 64), lambda b, i: (b, i, 0)),
            pl.BlockSpec((1, _RT, 8), lambda b, i: (b, i, 0)),
        ],
        out_shape=[
            jax.ShapeDtypeStruct((B, pos8.shape[1], 64), jnp.float32),
            jax.ShapeDtypeStruct((B, pos8.shape[1], 64), jnp.bfloat16),
            jax.ShapeDtypeStruct((B, pos8.shape[1], 8), jnp.float32),
        ],
    )(pos8, wposT, idx3, featx, w1x, w1m)

    if sharded:
        npl = _NP // jax.lax.axis_size('x')
        off = jax.lax.axis_index('x') * npl
        aux = jax.lax.dynamic_slice_in_dim(aux, off, npl, axis=1)
        h = jax.lax.dynamic_slice_in_dim(h, off, npl, axis=1)
        nt = npl // _RT

    outp = pl.pallas_call(
        _knn_kernel,
        grid=(B, nt),
        in_specs=[
            pl.BlockSpec((1, _RT, 8), lambda b, i: (b, i, 0)),
            pl.BlockSpec((1, 8, _NP // 128, 128), lambda b, i: (b, 0, 0, 0)),
            pl.BlockSpec((1, _NP, 64), lambda b, i: (b, 0, 0)),
            pl.BlockSpec((1, _RT, 64), lambda b, i: (b, i, 0)),
            pl.BlockSpec((64, 64), lambda b, i: (0, 0)),
            pl.BlockSpec((64, 64), lambda b, i: (0, 0)),
            pl.BlockSpec((64, 64), lambda b, i: (0, 0)),
            pl.BlockSpec((8, 8), lambda b, i: (0, 0)),
            pl.BlockSpec((64, 64), lambda b, i: (0, 0)),
            pl.BlockSpec((8, 16), lambda b, i: (0, 0)),
        ],
        out_specs=pl.BlockSpec((1, _RT, 16), lambda b, i: (b, i, 0)),
        out_shape=jax.ShapeDtypeStruct((B, aux.shape[1], 16), jnp.float32),
    )(aux, posT3, hb, h, Wm0, Wm, Wm2, wgp, wef, cst)
    return outp
